# baseline (speedup 1.0000x reference)
import functools

import jax
import jax.numpy as jnp
from jax import lax
from jax.experimental import pallas as pl
from jax.experimental.pallas import tpu as pltpu
from jax.experimental.pallas import tpu_sc as plsc

B, S, QW, OW, H, M, K = 8, 16, 2048, 2048, 16, 32768, 32
HW = QW // H
BS = B * S
CM = 2048
MC = M // CM

NC, NS, L = 2, 16, 16
NW = NC * NS
NR = H * BS
RW = NR // NW
NG = MC * CM // L
NGV = NG // L
NEG = -3.0e38


def _scores_body(x_ref, wq_ref, bq_ref, wk_ref, bk_ref, mem_ref, s_ref,
                 gm_ref, q_ref):
    mc = pl.program_id(1)

    @pl.when(mc == 0)
    def _():
        q_ref[...] = jnp.dot(x_ref[...], wq_ref[...],
                             preferred_element_type=jnp.float32) + bq_ref[0]

    keys = jnp.dot(mem_ref[0], wk_ref[0],
                   preferred_element_type=jnp.float32) + bk_ref[0]
    s = jax.lax.dot_general(
        q_ref[...], keys, (((1,), (1,)), ((), ())),
        preferred_element_type=jnp.float32)
    s_ref[0] = s
    m = jnp.maximum(s[:, :1024], s[:, 1024:])
    m = jnp.maximum(m[:, :512], m[:, 512:])
    m = jnp.maximum(m[:, :256], m[:, 256:])
    gm_ref[0] = jnp.maximum(m[:, :128], m[:, 128:])


NSPLIT = 4
HH = H // NSPLIT
NR2 = HH * BS
RW2 = NR2 // NW


def _scores(x2, memory, Wq, bq3, Wk, bk3, off):
    return pl.pallas_call(
        _scores_body,
        grid=(HH, MC),
        in_specs=[
            pl.BlockSpec((BS, QW), lambda h, mc: (0, 0)),
            pl.BlockSpec((QW, HW), lambda h, mc: (0, h + off)),
            pl.BlockSpec((1, 1, HW), lambda h, mc: (h + off, 0, 0)),
            pl.BlockSpec((1, HW, HW), lambda h, mc: (h + off, 0, 0)),
            pl.BlockSpec((1, 1, HW), lambda h, mc: (h + off, 0, 0)),
            pl.BlockSpec((1, CM, HW), lambda h, mc: (h + off, mc, 0)),
        ],
        out_specs=[
            pl.BlockSpec((1, BS, CM), lambda h, mc: (h, 0, mc)),
            pl.BlockSpec((1, BS, 128), lambda h, mc: (h, 0, mc)),
        ],
        out_shape=[
            jax.ShapeDtypeStruct((HH, BS, M), jnp.float32),
            jax.ShapeDtypeStruct((HH, BS, MC * 128), jnp.float32),
        ],
        scratch_shapes=[
            pltpu.VMEM((BS, HW), jnp.float32),
        ],
        compiler_params=pltpu.CompilerParams(
            dimension_semantics=("arbitrary", "arbitrary")),
    )(x2, Wq, bq3, Wk, bk3, memory)


def _merge16(xv, xi, t1v, t1i, t2v, t2i):
    rxv, rxi = lax.rev(xv, (0,)), lax.rev(xi, (0,))
    m = t1v >= rxv
    hv = jnp.where(m, t1v, rxv)
    hi = jnp.where(m, t1i, rxi)
    sv = jnp.where(m, rxv, t1v)
    si = jnp.where(m, rxi, t1i)
    hv, hi = plsc.sort_key_val(hv, hi)
    sv, si = plsc.sort_key_val(sv, si)
    rsv, rsi = lax.rev(sv, (0,)), lax.rev(si, (0,))
    m2 = t2v >= rsv
    lv = jnp.where(m2, t2v, rsv)
    li = jnp.where(m2, t2i, rsi)
    lv, li = plsc.sort_key_val(lv, li)
    return hv, hi, lv, li


def _make_topk_body(off):
  def _topk_body(gmax_hbm, scores1_hbm, memflat_hbm, out_hbm,
                 gm_a, gm_b, idx4_v, cand_v, ebase_v, midx_v, rows_v, acc_v,
                 sem, psem0, psem1):
    wid = lax.axis_index("s") * NC + lax.axis_index("c")
    base = wid * RW2
    iota = lax.iota(jnp.int32, L)
    ninf = jnp.full((L,), NEG, jnp.float32)
    zero = jnp.zeros((L,), jnp.int32)
    GB = 8
    psems = (psem0, psem1)
    gbufs = (gm_a, gm_b)

    pltpu.async_copy(gmax_hbm.at[base], gm_a, psem0)
    pltpu.async_copy(gmax_hbm.at[base + 1], gm_b, psem1)

    def do_row(i, gm_v, psem):
        r = base + i
        pltpu.make_async_copy(gmax_hbm.at[r], gm_v, psem).wait()

        def consider(xv, ebase, carry):
            t1v, t1i, t2v, t2i, tmin = carry
            xs, xis = plsc.sort_key_val(xv, ebase + iota)

            def yes(_):
                a, bi, c2, d = _merge16(xs, xis, t1v, t1i, t2v, t2i)
                return a, bi, c2, d, c2[0]

            def no(_):
                return carry

            return lax.cond(xs[L - 1] > tmin, yes, no, None)

        def gstep(g, carry):
            gb = pl.multiple_of(g * (GB * L), GB * L)
            xs = [gm_v[pl.ds(gb + u * L, L)] for u in range(GB)]
            m01 = jnp.maximum(xs[0], xs[1])
            m23 = jnp.maximum(xs[2], xs[3])
            m45 = jnp.maximum(xs[4], xs[5])
            m67 = jnp.maximum(xs[6], xs[7])
            m = jnp.maximum(jnp.maximum(m01, m23), jnp.maximum(m45, m67))
            ms, _ = plsc.sort_key_val(m, m)

            def hit(_):
                c = carry
                for u in range(GB):
                    c = consider(xs[u], gb + u * L, c)
                return c

            def miss(_):
                return carry

            return lax.cond(ms[L - 1] > carry[4], hit, miss, None)

        g1v, g1i, g2v, g2i, _ = lax.fori_loop(
            0, NGV // GB, gstep, (ninf, zero, ninf, zero, jnp.float32(NEG)))
        tstar = g2v[0]

        @pl.when(i + 2 < RW2)
        def _():
            pltpu.async_copy(gmax_hbm.at[r + 2], gm_v, psem)

        eb1 = (g1i >> 7) * CM + (g1i & 127)
        eb2 = (g2i >> 7) * CM + (g2i & 127)
        ebase_v[pl.ds(0, L)] = eb1
        ebase_v[pl.ds(L, L)] = eb2
        rM = r * M
        for t in range(L):
            for half, ebh in ((0, eb1), (1, eb2)):
                pos = (iota + half * L) * L + t
                val = rM + ebh + 128 * t
                plsc.store_scatter(idx4_v, [pos >> 7, pos & 127], val)
        descs = [pltpu.async_copy(scores1_hbm.at[idx4_v.at[j]],
                                  cand_v.at[j], sem) for j in range(4)]
        for d in descs:
            d.wait()

        c = (ninf, zero, ninf, zero, jnp.float32(NEG))
        for v in range(K):
            xv = cand_v[v // 8, pl.ds((v % 8) * L, L)]
            xs, xps = plsc.sort_key_val(xv, v * L + iota)

            def yes(_, xs=xs, xps=xps, c=c):
                t1v, t1i, t2v, t2i, _ = c
                a, bi, c2, d = _merge16(xs, xps, t1v, t1i, t2v, t2i)
                return a, bi, c2, d, c2[0]

            def no(_, c=c):
                return c

            c = lax.cond(xs[L - 1] >= jnp.maximum(c[4], tstar), yes, no, None)
        t1v, t1i, t2v, t2i, _ = c

        hm = (off + r // BS) * M
        s1 = plsc.load_gather(ebase_v, [t1i >> 4]) + 128 * (t1i & 15)
        s2 = plsc.load_gather(ebase_v, [t2i >> 4]) + 128 * (t2i & 15)
        midx_v[pl.ds(0, L)] = s1 + hm
        midx_v[pl.ds(L, L)] = s2 + hm
        g1 = 1.0 / (1.0 + jnp.exp(-t1v))
        g2 = 1.0 / (1.0 + jnp.exp(-t2v))
        pltpu.async_copy(memflat_hbm.at[midx_v], rows_v, sem).wait()
        accs = [jnp.zeros((L,), jnp.float32) for _ in range(HW // L)]
        for k in range(L):
            ga, gb = g1[k], g2[k]
            for jj in range(HW // L):
                accs[jj] = (accs[jj]
                            + rows_v[k, pl.ds(jj * L, L)] * ga
                            + rows_v[L + k, pl.ds(jj * L, L)] * gb)
        for jj in range(HW // L):
            acc_v[pl.ds(jj * L, L)] = accs[jj]
        pltpu.sync_copy(acc_v, out_hbm.at[r & (BS - 1), r // BS])

    def pair(p, _):
        for b in range(2):
            do_row(p * 2 + b, gbufs[b], psems[b])
        return 0

    lax.fori_loop(0, RW2 // 2, pair, 0)
  return _topk_body


def _topk_gather(gmax2, scores1, memflat, off):
    k = functools.partial(
        pl.kernel,
        out_type=jax.ShapeDtypeStruct((BS, HH, HW), jnp.float32),
        mesh=plsc.VectorSubcoreMesh(core_axis_name="c", subcore_axis_name="s"),
        scratch_types=[
            pltpu.VMEM((NG,), jnp.float32),
            pltpu.VMEM((NG,), jnp.float32),
            pltpu.VMEM((4, 128), jnp.int32),
            pltpu.VMEM((4, 128), jnp.float32),
            pltpu.VMEM((K,), jnp.int32),
            pltpu.VMEM((K,), jnp.int32),
            pltpu.VMEM((K, HW), jnp.float32),
            pltpu.VMEM((HW,), jnp.float32),
            pltpu.SemaphoreType.DMA,
            pltpu.SemaphoreType.DMA,
            pltpu.SemaphoreType.DMA,
        ],
        compiler_params=pltpu.CompilerParams(needs_layout_passes=False),
    )(_make_topk_body(off))
    return k(gmax2, scores1, memflat)


def _proj_body(x_ref, w_ref, b_ref, o_ref):
    o_ref[...] = jnp.dot(x_ref[...], w_ref[...],
                         preferred_element_type=jnp.float32) + b_ref[0][None, :]


def _final_proj(out2, Wo, bo2):
    CO = 512
    return pl.pallas_call(
        _proj_body,
        grid=(OW // CO,),
        in_specs=[
            pl.BlockSpec((BS, QW), lambda j: (0, 0)),
            pl.BlockSpec((QW, CO), lambda j: (0, j)),
            pl.BlockSpec((1, CO), lambda j: (0, j)),
        ],
        out_specs=pl.BlockSpec((BS, CO), lambda j: (0, j)),
        out_shape=jax.ShapeDtypeStruct((BS, OW), jnp.float32),
    )(out2, Wo, bo2)


def kernel(tensor, memory, Wq, bq, Wk, bk, Wo, bo):
    x2 = tensor.reshape(BS, QW)
    bq3 = bq.reshape(H, 1, HW)
    bk3 = bk.reshape(H, 1, HW)
    memflat = memory.reshape(H * M, HW)

    mids = []
    for half in range(NSPLIT):
        off = half * HH
        scores, gmax = _scores(x2, memory, Wq, bq3, Wk, bk3, off)
        mids.append(_topk_gather(gmax.reshape(NR2, MC * 128),
                                 scores.reshape(NR2 * M), memflat, off))
    out2 = jnp.concatenate(mids, axis=1).reshape(BS, H * HW)

    return _final_proj(out2, Wo, bo.reshape(1, OW)).reshape(B, S, OW)


# speedup vs baseline: 1.0078x; 1.0078x over previous
import functools

import jax
import jax.numpy as jnp
from jax import lax
from jax.experimental import pallas as pl
from jax.experimental.pallas import tpu as pltpu
from jax.experimental.pallas import tpu_sc as plsc

B, S, QW, OW, H, M, K = 8, 16, 2048, 2048, 16, 32768, 32
HW = QW // H
BS = B * S
CM = 2048
MC = M // CM

NC, NS, L = 2, 16, 16
NW = NC * NS
NR = H * BS
RW = NR // NW
NG = MC * CM // L
NGV = NG // L
NEG = -3.0e38


def _scores_body(x_ref, wq_ref, bq_ref, wk_ref, bk_ref, mem_ref, s_ref,
                 gm_ref, q_ref):
    mc = pl.program_id(1)

    @pl.when(mc == 0)
    def _():
        q_ref[...] = jnp.dot(x_ref[...], wq_ref[...],
                             preferred_element_type=jnp.float32) + bq_ref[0]

    keys = jnp.dot(mem_ref[0], wk_ref[0],
                   preferred_element_type=jnp.float32) + bk_ref[0]
    s = jax.lax.dot_general(
        q_ref[...], keys, (((1,), (1,)), ((), ())),
        preferred_element_type=jnp.float32)
    s_ref[0] = s
    m = jnp.maximum(s[:, :1024], s[:, 1024:])
    m = jnp.maximum(m[:, :512], m[:, 512:])
    m = jnp.maximum(m[:, :256], m[:, 256:])
    gm_ref[0] = jnp.maximum(m[:, :128], m[:, 128:])


NSPLIT = 2
HH = H // NSPLIT
NR2 = HH * BS
RW2 = NR2 // NW


def _scores(x2, memory, Wq, bq3, Wk, bk3, off):
    return pl.pallas_call(
        _scores_body,
        grid=(HH, MC),
        in_specs=[
            pl.BlockSpec((BS, QW), lambda h, mc: (0, 0)),
            pl.BlockSpec((QW, HW), lambda h, mc: (0, h + off)),
            pl.BlockSpec((1, 1, HW), lambda h, mc: (h + off, 0, 0)),
            pl.BlockSpec((1, HW, HW), lambda h, mc: (h + off, 0, 0)),
            pl.BlockSpec((1, 1, HW), lambda h, mc: (h + off, 0, 0)),
            pl.BlockSpec((1, CM, HW), lambda h, mc: (h + off, mc, 0)),
        ],
        out_specs=[
            pl.BlockSpec((1, BS, CM), lambda h, mc: (h, 0, mc)),
            pl.BlockSpec((1, BS, 128), lambda h, mc: (h, 0, mc)),
        ],
        out_shape=[
            jax.ShapeDtypeStruct((HH, BS, M), jnp.float32),
            jax.ShapeDtypeStruct((HH, BS, MC * 128), jnp.float32),
        ],
        scratch_shapes=[
            pltpu.VMEM((BS, HW), jnp.float32),
        ],
        compiler_params=pltpu.CompilerParams(
            dimension_semantics=("arbitrary", "arbitrary")),
    )(x2, Wq, bq3, Wk, bk3, memory)


def _merge16(xv, xi, t1v, t1i, t2v, t2i):
    rxv, rxi = lax.rev(xv, (0,)), lax.rev(xi, (0,))
    m = t1v >= rxv
    hv = jnp.where(m, t1v, rxv)
    hi = jnp.where(m, t1i, rxi)
    sv = jnp.where(m, rxv, t1v)
    si = jnp.where(m, rxi, t1i)
    hv, hi = plsc.sort_key_val(hv, hi)
    sv, si = plsc.sort_key_val(sv, si)
    rsv, rsi = lax.rev(sv, (0,)), lax.rev(si, (0,))
    m2 = t2v >= rsv
    lv = jnp.where(m2, t2v, rsv)
    li = jnp.where(m2, t2i, rsi)
    lv, li = plsc.sort_key_val(lv, li)
    return hv, hi, lv, li


def _make_topk_body(off):
  def _topk_body(gmax_hbm, scores1_hbm, memflat_hbm, out_hbm,
                 gm_a, gm_b, idx4_v, cand_v, ebase_v, midx_v, rows_v, acc_v,
                 sem, psem0, psem1):
    wid = lax.axis_index("s") * NC + lax.axis_index("c")
    base = wid * RW2
    iota = lax.iota(jnp.int32, L)
    ninf = jnp.full((L,), NEG, jnp.float32)
    zero = jnp.zeros((L,), jnp.int32)
    GB = 8
    psems = (psem0, psem1)
    gbufs = (gm_a, gm_b)

    pltpu.async_copy(gmax_hbm.at[base], gm_a, psem0)
    pltpu.async_copy(gmax_hbm.at[base + 1], gm_b, psem1)

    def do_row(i, gm_v, psem):
        r = base + i
        pltpu.make_async_copy(gmax_hbm.at[r], gm_v, psem).wait()

        def consider(xv, ebase, carry):
            t1v, t1i, t2v, t2i, tmin = carry
            xs, xis = plsc.sort_key_val(xv, ebase + iota)

            def yes(_):
                a, bi, c2, d = _merge16(xs, xis, t1v, t1i, t2v, t2i)
                return a, bi, c2, d, c2[0]

            def no(_):
                return carry

            return lax.cond(xs[L - 1] > tmin, yes, no, None)

        def gstep(g, carry):
            gb = pl.multiple_of(g * (GB * L), GB * L)
            xs = [gm_v[pl.ds(gb + u * L, L)] for u in range(GB)]
            m01 = jnp.maximum(xs[0], xs[1])
            m23 = jnp.maximum(xs[2], xs[3])
            m45 = jnp.maximum(xs[4], xs[5])
            m67 = jnp.maximum(xs[6], xs[7])
            m = jnp.maximum(jnp.maximum(m01, m23), jnp.maximum(m45, m67))
            ms, _ = plsc.sort_key_val(m, m)

            def hit(_):
                c = carry
                for u in range(GB):
                    c = consider(xs[u], gb + u * L, c)
                return c

            def miss(_):
                return carry

            return lax.cond(ms[L - 1] > carry[4], hit, miss, None)

        g1v, g1i, g2v, g2i, _ = lax.fori_loop(
            0, NGV // GB, gstep, (ninf, zero, ninf, zero, jnp.float32(NEG)))
        tstar = g2v[0]

        @pl.when(i + 2 < RW2)
        def _():
            pltpu.async_copy(gmax_hbm.at[r + 2], gm_v, psem)

        eb1 = (g1i >> 7) * CM + (g1i & 127)
        eb2 = (g2i >> 7) * CM + (g2i & 127)
        ebase_v[pl.ds(0, L)] = eb1
        ebase_v[pl.ds(L, L)] = eb2
        rM = r * M
        for t in range(L):
            for half, ebh in ((0, eb1), (1, eb2)):
                pos = (iota + half * L) * L + t
                val = rM + ebh + 128 * t
                plsc.store_scatter(idx4_v, [pos >> 7, pos & 127], val)
        descs = [pltpu.async_copy(scores1_hbm.at[idx4_v.at[j]],
                                  cand_v.at[j], sem) for j in range(4)]
        for d in descs:
            d.wait()

        c = (ninf, zero, ninf, zero, jnp.float32(NEG))
        for v in range(K):
            xv = cand_v[v // 8, pl.ds((v % 8) * L, L)]
            xs, xps = plsc.sort_key_val(xv, v * L + iota)

            def yes(_, xs=xs, xps=xps, c=c):
                t1v, t1i, t2v, t2i, _ = c
                a, bi, c2, d = _merge16(xs, xps, t1v, t1i, t2v, t2i)
                return a, bi, c2, d, c2[0]

            def no(_, c=c):
                return c

            c = lax.cond(xs[L - 1] >= jnp.maximum(c[4], tstar), yes, no, None)
        t1v, t1i, t2v, t2i, _ = c

        hm = (off + r // BS) * M
        s1 = plsc.load_gather(ebase_v, [t1i >> 4]) + 128 * (t1i & 15)
        s2 = plsc.load_gather(ebase_v, [t2i >> 4]) + 128 * (t2i & 15)
        midx_v[pl.ds(0, L)] = s1 + hm
        midx_v[pl.ds(L, L)] = s2 + hm
        g1 = 1.0 / (1.0 + jnp.exp(-t1v))
        g2 = 1.0 / (1.0 + jnp.exp(-t2v))
        pltpu.async_copy(memflat_hbm.at[midx_v], rows_v, sem).wait()
        accs = [jnp.zeros((L,), jnp.float32) for _ in range(HW // L)]
        for k in range(L):
            ga, gb = g1[k], g2[k]
            for jj in range(HW // L):
                accs[jj] = (accs[jj]
                            + rows_v[k, pl.ds(jj * L, L)] * ga
                            + rows_v[L + k, pl.ds(jj * L, L)] * gb)
        for jj in range(HW // L):
            acc_v[pl.ds(jj * L, L)] = accs[jj]
        pltpu.sync_copy(acc_v, out_hbm.at[r & (BS - 1), r // BS])

    def pair(p, _):
        for b in range(2):
            do_row(p * 2 + b, gbufs[b], psems[b])
        return 0

    lax.fori_loop(0, RW2 // 2, pair, 0)
  return _topk_body


def _topk_gather(gmax2, scores1, memflat, off):
    k = functools.partial(
        pl.kernel,
        out_type=jax.ShapeDtypeStruct((BS, HH, HW), jnp.float32),
        mesh=plsc.VectorSubcoreMesh(core_axis_name="c", subcore_axis_name="s"),
        scratch_types=[
            pltpu.VMEM((NG,), jnp.float32),
            pltpu.VMEM((NG,), jnp.float32),
            pltpu.VMEM((4, 128), jnp.int32),
            pltpu.VMEM((4, 128), jnp.float32),
            pltpu.VMEM((K,), jnp.int32),
            pltpu.VMEM((K,), jnp.int32),
            pltpu.VMEM((K, HW), jnp.float32),
            pltpu.VMEM((HW,), jnp.float32),
            pltpu.SemaphoreType.DMA,
            pltpu.SemaphoreType.DMA,
            pltpu.SemaphoreType.DMA,
        ],
        compiler_params=pltpu.CompilerParams(needs_layout_passes=False),
    )(_make_topk_body(off))
    return k(gmax2, scores1, memflat)


def _proj_body(x_ref, w_ref, b_ref, o_ref):
    o_ref[...] = jnp.dot(x_ref[...], w_ref[...],
                         preferred_element_type=jnp.float32) + b_ref[0][None, :]


def _final_proj(out2, Wo, bo2):
    CO = 512
    return pl.pallas_call(
        _proj_body,
        grid=(OW // CO,),
        in_specs=[
            pl.BlockSpec((BS, QW), lambda j: (0, 0)),
            pl.BlockSpec((QW, CO), lambda j: (0, j)),
            pl.BlockSpec((1, CO), lambda j: (0, j)),
        ],
        out_specs=pl.BlockSpec((BS, CO), lambda j: (0, j)),
        out_shape=jax.ShapeDtypeStruct((BS, OW), jnp.float32),
    )(out2, Wo, bo2)


def kernel(tensor, memory, Wq, bq, Wk, bk, Wo, bo):
    x2 = tensor.reshape(BS, QW)
    bq3 = bq.reshape(H, 1, HW)
    bk3 = bk.reshape(H, 1, HW)
    memflat = memory.reshape(H * M, HW)

    mids = []
    for half in range(NSPLIT):
        off = half * HH
        scores, gmax = _scores(x2, memory, Wq, bq3, Wk, bk3, off)
        mids.append(_topk_gather(gmax.reshape(NR2, MC * 128),
                                 scores.reshape(NR2 * M), memflat, off))
    out2 = jnp.concatenate(mids, axis=1).reshape(BS, H * HW)

    return _final_proj(out2, Wo, bo.reshape(1, OW)).reshape(B, S, OW)
